# Initial kernel scaffold; baseline (speedup 1.0000x reference)
import functools

import jax
import jax.numpy as jnp
from jax import lax
from jax.experimental import pallas as pl
from jax.experimental.pallas import tpu as pltpu
from jax.experimental.pallas import tpu_sc as plsc

N = 10000
E = 320000
D = 128
H = 128
FC = 32

NC = 2
NS = 16
NW = NC * NS
EPW = E // NW
CH = 128
NFULL = EPW // CH
TAIL = EPW - NFULL * CH

SLAB = N // NS

_MESH = plsc.VectorSubcoreMesh(
    core_axis_name="c", subcore_axis_name="s", num_cores=NC, num_subcores=NS
)


def _worker(c, s):
  return c * NS + s


def _deg_body(dst_hbm, out_hbm, idx_v, idxt_v, ones_v, onest_v, zer_v, acc_sh):
  c = lax.axis_index("c")
  s = lax.axis_index("s")
  base = _worker(c, s) * EPW

  def fill(i, _):
    ones_v[pl.ds(i * 16, 16)] = jnp.full((16,), 1.0, jnp.float32)
    return 0

  lax.fori_loop(0, CH // 16, fill, 0)
  onest_v[...] = jnp.full((TAIL,), 1.0, jnp.float32)

  def fillz(i, _):
    zer_v[pl.ds(i * 16, 16)] = jnp.zeros((16,), jnp.float32)
    return 0

  lax.fori_loop(0, 640 // 16, fillz, 0)

  @pl.when(s < NS - 1)
  def _():
    pltpu.sync_copy(zer_v.at[pl.ds(0, 624)], acc_sh.at[pl.ds(s * 624, 624)])

  @pl.when(s == NS - 1)
  def _():
    pltpu.sync_copy(zer_v, acc_sh.at[pl.ds(624 * (NS - 1), 640)])

  plsc.subcore_barrier()

  def body(j, _):
    pltpu.sync_copy(dst_hbm.at[pl.ds(base + j * CH, CH)], idx_v)
    pltpu.sync_copy(ones_v, acc_sh.at[idx_v], add=True)
    return 0

  lax.fori_loop(0, NFULL, body, 0)
  pltpu.sync_copy(dst_hbm.at[pl.ds(base + NFULL * CH, TAIL)], idxt_v)
  pltpu.sync_copy(onest_v, acc_sh.at[idxt_v], add=True)

  plsc.subcore_barrier()

  @pl.when(s < NS - 1)
  def _():
    pltpu.sync_copy(acc_sh.at[pl.ds(s * 624, 624)], out_hbm.at[c, pl.ds(s * 624, 624)])

  @pl.when(s == NS - 1)
  def _():
    pltpu.sync_copy(acc_sh.at[pl.ds(624 * (NS - 1), 640)],
                    out_hbm.at[c, pl.ds(624 * (NS - 1), 640)])


_deg_call = pl.kernel(
    _deg_body,
    out_type=jax.ShapeDtypeStruct((NC, N), jnp.float32),
    mesh=_MESH,
    scratch_types=[
        pltpu.VMEM((CH,), jnp.int32),
        pltpu.VMEM((TAIL,), jnp.int32),
        pltpu.VMEM((CH,), jnp.float32),
        pltpu.VMEM((TAIL,), jnp.float32),
        pltpu.VMEM((640,), jnp.float32),
        pltpu.VMEM_SHARED((N,), jnp.float32),
    ],
)


def _agg_body(t_hbm, src_hbm, dst_hbm, out_hbm,
              si_v, di_v, rows_v, sit_v, dit_v, rowst_v, acc_sh, sem):
  c = lax.axis_index("c")
  s = lax.axis_index("s")
  base = _worker(c, s) * EPW
  row0 = s * SLAB

  def fz(i, _):
    rows_v[i // 8, pl.ds((i % 8) * 16, 16)] = jnp.zeros((16,), jnp.float32)
    return 0

  lax.fori_loop(0, CH * (D // 16), fz, 0)
  for k in range(5):
    pltpu.sync_copy(rows_v.at[pl.ds(0, 125)],
                    acc_sh.at[pl.ds(row0 + 125 * k, 125)])
  plsc.subcore_barrier()

  def body(j, _):
    pltpu.sync_copy(src_hbm.at[pl.ds(base + j * CH, CH)], si_v)
    pltpu.sync_copy(dst_hbm.at[pl.ds(base + j * CH, CH)], di_v)
    pltpu.async_copy(t_hbm.at[si_v], rows_v, sem).wait()
    pltpu.sync_copy(rows_v, acc_sh.at[di_v], add=True)
    return 0

  lax.fori_loop(0, NFULL, body, 0)
  pltpu.sync_copy(src_hbm.at[pl.ds(base + NFULL * CH, TAIL)], sit_v)
  pltpu.sync_copy(dst_hbm.at[pl.ds(base + NFULL * CH, TAIL)], dit_v)
  pltpu.async_copy(t_hbm.at[sit_v], rowst_v, sem).wait()
  pltpu.sync_copy(rowst_v, acc_sh.at[dit_v], add=True)

  plsc.subcore_barrier()
  for k in range(5):
    pltpu.sync_copy(acc_sh.at[pl.ds(row0 + 125 * k, 125)],
                    out_hbm.at[c, pl.ds(row0 + 125 * k, 125)])


_agg_call = pl.kernel(
    _agg_body,
    out_type=jax.ShapeDtypeStruct((NC, N, D), jnp.float32),
    mesh=_MESH,
    scratch_types=[
        pltpu.VMEM((CH,), jnp.int32),
        pltpu.VMEM((CH,), jnp.int32),
        pltpu.VMEM((CH, D), jnp.float32),
        pltpu.VMEM((TAIL,), jnp.int32),
        pltpu.VMEM((TAIL,), jnp.int32),
        pltpu.VMEM((TAIL, D), jnp.float32),
        pltpu.VMEM_SHARED((N, D), jnp.float32),
        pltpu.SemaphoreType.DMA,
    ],
)


RB = 2000
_GRID = N // RB


def _rows(i):
  return (i, 0)


def _full(i):
  return (0, 0)


def _dis(d0_ref, d1_ref):
  return lax.rsqrt(1.0 + d0_ref[...] + d1_ref[...])


def _mm1_body(x_ref, w_ref, d0_ref, d1_ref, o_ref):
  o_ref[...] = jnp.dot(x_ref[...], w_ref[...],
                       preferred_element_type=jnp.float32) * _dis(d0_ref, d1_ref)


def _mm2_body(p0_ref, p1_ref, t_ref, d0_ref, d1_ref, b_ref, w_ref, o_ref):
  dis = _dis(d0_ref, d1_ref)
  h = jnp.maximum(dis * (p0_ref[...] + p1_ref[...] + t_ref[...]) + b_ref[...], 0.0)
  o_ref[...] = jnp.dot(h, w_ref[...], preferred_element_type=jnp.float32) * dis


def _head_body(p0_ref, p1_ref, t_ref, d0_ref, d1_ref, b_ref,
               wf_ref, bf_ref, wo_ref, bo_ref, o_ref):
  dis = _dis(d0_ref, d1_ref)
  h2 = jnp.maximum(dis * (p0_ref[...] + p1_ref[...] + t_ref[...]) + b_ref[...], 0.0)
  h3 = jnp.maximum(jnp.dot(h2, wf_ref[...], preferred_element_type=jnp.float32)
                   + bf_ref[...], 0.0)
  o_ref[...] = jnp.dot(h3, wo_ref[...], preferred_element_type=jnp.float32) + bo_ref[...]


def _row_spec(cols):
  return pl.BlockSpec((RB, cols), _rows)


def _w_spec(r, c):
  return pl.BlockSpec((r, c), _full)


_mm1 = pl.pallas_call(
    _mm1_body,
    grid=(_GRID,),
    in_specs=[_row_spec(D), _w_spec(D, H), _row_spec(1), _row_spec(1)],
    out_specs=_row_spec(H),
    out_shape=jax.ShapeDtypeStruct((N, H), jnp.float32),
)

_mm2 = pl.pallas_call(
    _mm2_body,
    grid=(_GRID,),
    in_specs=[_row_spec(H), _row_spec(H), _row_spec(H), _row_spec(1),
              _row_spec(1), _w_spec(1, H), _w_spec(H, H)],
    out_specs=_row_spec(H),
    out_shape=jax.ShapeDtypeStruct((N, H), jnp.float32),
)

_head = pl.pallas_call(
    _head_body,
    grid=(_GRID,),
    in_specs=[_row_spec(H), _row_spec(H), _row_spec(H), _row_spec(1),
              _row_spec(1), _w_spec(1, H), _w_spec(H, FC), _w_spec(1, FC),
              _w_spec(FC, 1), _w_spec(1, 1)],
    out_specs=_row_spec(1),
    out_shape=jax.ShapeDtypeStruct((N, 1), jnp.float32),
)


def kernel(x, edge_index, W1, b1, W2, b2, Wf, bf, Wo, bo):
  src = edge_index[0]
  dst = edge_index[1]

  degp = _deg_call(dst)
  d0 = degp[0][:, None]
  d1 = degp[1][:, None]

  t1 = _mm1(x, W1, d0, d1)
  p = _agg_call(t1, src, dst)
  t2 = _mm2(p[0], p[1], t1, d0, d1, b1[None, :], W2)
  q = _agg_call(t2, src, dst)
  return _head(q[0], q[1], t2, d0, d1, b2[None, :],
               Wf, bf[None, :], Wo, bo[None, :])


# speedup vs baseline: 16.2114x; 16.2114x over previous
import jax
import jax.numpy as jnp
from jax import lax
from jax.experimental import pallas as pl
from jax.experimental.pallas import tpu as pltpu
from jax.experimental.pallas import tpu_sc as plsc

N = 10000
E = 320000
D = 128
H = 128
FC = 32

NC = 2
NS = 16
NW = NC * NS
EPW = E // NW
CH = 128
NFULL = EPW // CH
TAIL = EPW - NFULL * CH

SLAB = 640
LAST = N - (NS - 1) * SLAB
NP = NS * SLAB

_MESH = plsc.VectorSubcoreMesh(
    core_axis_name="c", subcore_axis_name="s", num_cores=NC, num_subcores=NS
)


def _deg_body(dst_hbm, out0_hbm, out1_hbm, idx_v, idxt_v, ones_v, onest_v,
              zer_v, acc_sh):
  c = lax.axis_index("c")
  s = lax.axis_index("s")
  base = (c * NS + s) * EPW

  def fill(i, _):
    ones_v[pl.ds(i * 16, 16)] = jnp.full((16,), 1.0, jnp.float32)
    return 0

  lax.fori_loop(0, CH // 16, fill, 0)
  onest_v[...] = jnp.full((TAIL,), 1.0, jnp.float32)

  def fillz(i, _):
    zer_v[pl.ds(i * 16, 16)] = jnp.zeros((16,), jnp.float32)
    return 0

  lax.fori_loop(0, SLAB // 16, fillz, 0)

  pltpu.sync_copy(zer_v, acc_sh.at[pl.ds(s * SLAB, SLAB)])

  plsc.subcore_barrier()

  def body(j, _):
    pltpu.sync_copy(dst_hbm.at[pl.ds(base + j * CH, CH)], idx_v)
    pltpu.sync_copy(ones_v, acc_sh.at[idx_v], add=True)
    return 0

  lax.fori_loop(0, NFULL, body, 0)
  pltpu.sync_copy(dst_hbm.at[pl.ds(base + NFULL * CH, TAIL)], idxt_v)
  pltpu.sync_copy(onest_v, acc_sh.at[idxt_v], add=True)

  plsc.subcore_barrier()

  @pl.when(c == 0)
  def _():
    pltpu.sync_copy(acc_sh.at[pl.ds(s * SLAB, SLAB)],
                    out0_hbm.at[pl.ds(s * SLAB, SLAB)])

  @pl.when(c == 1)
  def _():
    pltpu.sync_copy(acc_sh.at[pl.ds(s * SLAB, SLAB)],
                    out1_hbm.at[pl.ds(s * SLAB, SLAB)])


_deg_call = pl.kernel(
    _deg_body,
    out_type=(jax.ShapeDtypeStruct((NP,), jnp.float32),
              jax.ShapeDtypeStruct((NP,), jnp.float32)),
    mesh=_MESH,
    scratch_types=[
        pltpu.VMEM((CH,), jnp.int32),
        pltpu.VMEM((TAIL,), jnp.int32),
        pltpu.VMEM((CH,), jnp.float32),
        pltpu.VMEM((TAIL,), jnp.float32),
        pltpu.VMEM((SLAB,), jnp.float32),
        pltpu.VMEM_SHARED((NP,), jnp.float32),
    ],
)


def _agg_body(t_hbm, src_hbm, dst_hbm, out0_hbm, out1_hbm,
              si_v, di_v, rows_v, sit_v, dit_v, rowst_v, acc_sh, sem):
  c = lax.axis_index("c")
  s = lax.axis_index("s")
  base = (c * NS + s) * EPW
  row0 = s * SLAB

  def fz(i, _):
    rows_v[i // 8, pl.ds((i % 8) * 16, 16)] = jnp.zeros((16,), jnp.float32)
    return 0

  lax.fori_loop(0, CH * (D // 16), fz, 0)

  @pl.when(s < NS - 1)
  def _():
    for k in range(SLAB // CH):
      pltpu.sync_copy(rows_v, acc_sh.at[pl.ds(row0 + CH * k, CH)])

  @pl.when(s == NS - 1)
  def _():
    for k in range(LAST // CH):
      pltpu.sync_copy(rows_v, acc_sh.at[pl.ds(row0 + CH * k, CH)])
    pltpu.sync_copy(rows_v.at[pl.ds(0, LAST % CH)],
                    acc_sh.at[pl.ds(row0 + (LAST // CH) * CH, LAST % CH)])

  plsc.subcore_barrier()

  def body(j, _):
    pltpu.sync_copy(src_hbm.at[pl.ds(base + j * CH, CH)], si_v)
    pltpu.sync_copy(dst_hbm.at[pl.ds(base + j * CH, CH)], di_v)
    pltpu.async_copy(t_hbm.at[si_v], rows_v, sem).wait()
    pltpu.sync_copy(rows_v, acc_sh.at[di_v], add=True)
    return 0

  lax.fori_loop(0, NFULL, body, 0)
  pltpu.sync_copy(src_hbm.at[pl.ds(base + NFULL * CH, TAIL)], sit_v)
  pltpu.sync_copy(dst_hbm.at[pl.ds(base + NFULL * CH, TAIL)], dit_v)
  pltpu.async_copy(t_hbm.at[sit_v], rowst_v, sem).wait()
  pltpu.sync_copy(rowst_v, acc_sh.at[dit_v], add=True)

  plsc.subcore_barrier()

  def writeback(out_ref):
    @pl.when(s < NS - 1)
    def _():
      for k in range(SLAB // CH):
        pltpu.sync_copy(acc_sh.at[pl.ds(row0 + CH * k, CH)],
                        out_ref.at[pl.ds(row0 + CH * k, CH)])

    @pl.when(s == NS - 1)
    def _():
      for k in range(LAST // CH):
        pltpu.sync_copy(acc_sh.at[pl.ds(row0 + CH * k, CH)],
                        out_ref.at[pl.ds(row0 + CH * k, CH)])
      pltpu.sync_copy(acc_sh.at[pl.ds(row0 + (LAST // CH) * CH, LAST % CH)],
                      out_ref.at[pl.ds(row0 + (LAST // CH) * CH, LAST % CH)])

  @pl.when(c == 0)
  def _():
    writeback(out0_hbm)

  @pl.when(c == 1)
  def _():
    writeback(out1_hbm)


_agg_call = pl.kernel(
    _agg_body,
    out_type=(jax.ShapeDtypeStruct((N, D), jnp.float32),
              jax.ShapeDtypeStruct((N, D), jnp.float32)),
    mesh=_MESH,
    scratch_types=[
        pltpu.VMEM((CH,), jnp.int32),
        pltpu.VMEM((CH,), jnp.int32),
        pltpu.VMEM((CH, D), jnp.float32),
        pltpu.VMEM((TAIL,), jnp.int32),
        pltpu.VMEM((TAIL,), jnp.int32),
        pltpu.VMEM((TAIL, D), jnp.float32),
        pltpu.VMEM_SHARED((N, D), jnp.float32),
        pltpu.SemaphoreType.DMA,
    ],
)


RB = 2000
_GRID = N // RB


def _rows(i):
  return (i, 0)


def _full(i):
  return (0, 0)


def _dis(d0_ref, d1_ref):
  return lax.rsqrt(1.0 + d0_ref[...] + d1_ref[...])


def _mm1_body(x_ref, w_ref, d0_ref, d1_ref, o_ref):
  o_ref[...] = jnp.dot(x_ref[...], w_ref[...],
                       preferred_element_type=jnp.float32) * _dis(d0_ref, d1_ref)


def _mm2_body(p0_ref, p1_ref, t_ref, d0_ref, d1_ref, b_ref, w_ref, o_ref):
  dis = _dis(d0_ref, d1_ref)
  h = jnp.maximum(dis * (p0_ref[...] + p1_ref[...] + t_ref[...]) + b_ref[...], 0.0)
  o_ref[...] = jnp.dot(h, w_ref[...], preferred_element_type=jnp.float32) * dis


def _head_body(p0_ref, p1_ref, t_ref, d0_ref, d1_ref, b_ref,
               wf_ref, bf_ref, wo_ref, bo_ref, o_ref):
  dis = _dis(d0_ref, d1_ref)
  h2 = jnp.maximum(dis * (p0_ref[...] + p1_ref[...] + t_ref[...]) + b_ref[...], 0.0)
  h3 = jnp.maximum(jnp.dot(h2, wf_ref[...], preferred_element_type=jnp.float32)
                   + bf_ref[...], 0.0)
  o_ref[...] = jnp.dot(h3, wo_ref[...], preferred_element_type=jnp.float32) + bo_ref[...]


def _row_spec(cols):
  return pl.BlockSpec((RB, cols), _rows)


def _w_spec(r, c):
  return pl.BlockSpec((r, c), _full)


_mm1 = pl.pallas_call(
    _mm1_body,
    grid=(_GRID,),
    in_specs=[_row_spec(D), _w_spec(D, H), _row_spec(1), _row_spec(1)],
    out_specs=_row_spec(H),
    out_shape=jax.ShapeDtypeStruct((N, H), jnp.float32),
)

_mm2 = pl.pallas_call(
    _mm2_body,
    grid=(_GRID,),
    in_specs=[_row_spec(H), _row_spec(H), _row_spec(H), _row_spec(1),
              _row_spec(1), _w_spec(1, H), _w_spec(H, H)],
    out_specs=_row_spec(H),
    out_shape=jax.ShapeDtypeStruct((N, H), jnp.float32),
)

_head = pl.pallas_call(
    _head_body,
    grid=(_GRID,),
    in_specs=[_row_spec(H), _row_spec(H), _row_spec(H), _row_spec(1),
              _row_spec(1), _w_spec(1, H), _w_spec(H, FC), _w_spec(1, FC),
              _w_spec(FC, 1), _w_spec(1, 1)],
    out_specs=_row_spec(1),
    out_shape=jax.ShapeDtypeStruct((N, 1), jnp.float32),
)


def kernel(x, edge_index, W1, b1, W2, b2, Wf, bf, Wo, bo):
  src = edge_index[0]
  dst = edge_index[1]

  deg0, deg1 = _deg_call(dst)
  d0 = deg0[:N, None]
  d1 = deg1[:N, None]

  t1 = _mm1(x, W1, d0, d1)
  p0, p1 = _agg_call(t1, src, dst)
  t2 = _mm2(p0, p1, t1, d0, d1, b1[None, :], W2)
  q0, q1 = _agg_call(t2, src, dst)
  return _head(q0, q1, t2, d0, d1, b2[None, :],
               Wf, bf[None, :], Wo, bo[None, :])
